# baseline (speedup 1.0000x reference)
import functools

import jax
import jax.numpy as jnp
from jax import lax
from jax.experimental import pallas as pl
from jax.experimental.pallas import tpu as pltpu
from jax.experimental.pallas import tpu_sc as plsc

N = 50000
D = 128
E = 500000
CW = 64
NCH = D // CW
EPAD = 524288
NSUB = 16
EPT = EPAD // NSUB
MICRO = 128
QROWS = 32
IDROWS = EPAD // MICRO
TROWS = EPT // MICRO
NPAD = 50048
ACC_ROWS = NPAD
WPT = NPAD // NSUB
DEG_ROWS = NPAD
DPT = DEG_ROWS // NSUB
BM = 2000


def _y_body(xu_ref, xi_ref, wui_ref, wiu_ref, wuu_ref, yui_ref, yiu_ref, yuu_ref):
    xu = xu_ref[...]
    xi = xi_ref[...]
    yui = jnp.dot(xu, wui_ref[...], preferred_element_type=jnp.float32)
    yiu = jnp.dot(xi, wiu_ref[...], preferred_element_type=jnp.float32)
    yuu = jnp.dot(xu, wuu_ref[...], preferred_element_type=jnp.float32)
    for c in range(NCH):
        yui_ref[c] = yui[:, c * CW:(c + 1) * CW].astype(jnp.bfloat16)
        yiu_ref[c] = yiu[:, c * CW:(c + 1) * CW].astype(jnp.bfloat16)
        yuu_ref[c] = yuu[:, c * CW:(c + 1) * CW].astype(jnp.bfloat16)


def _y_matmuls(x_user, x_item, w_ui, w_iu, w_uu):
    xspec = pl.BlockSpec((BM, D), lambda r: (r, 0))
    wspec = pl.BlockSpec((D, D), lambda r: (0, 0))
    yspec = pl.BlockSpec((NCH, BM, CW), lambda r: (0, r, 0))
    yshape = jax.ShapeDtypeStruct((NCH, N, CW), jnp.bfloat16)
    return pl.pallas_call(
        _y_body,
        grid=(N // BM,),
        in_specs=[xspec, xspec, wspec, wspec, wspec],
        out_specs=[yspec, yspec, yspec],
        out_shape=[yshape, yshape, yshape],
    )(x_user, x_item, w_ui, w_iu, w_uu)


def _combine_body(aui_ref, aiu_ref, auu_ref, dui_ref, diu_ref, duu_ref,
                  xu_ref, xi_ref, wdui_ref, wdiu_ref, wduu_ref,
                  bui_ref, biu_ref, buu_ref, ou_ref, oi_ref):
    rui = 1.0 / jnp.maximum(dui_ref[0] + dui_ref[1], 1.0)
    riu = 1.0 / jnp.maximum(diu_ref[0] + diu_ref[1], 1.0)
    ruu = 1.0 / jnp.maximum(duu_ref[0] + duu_ref[1], 1.0)
    xu = xu_ref[...]
    xi = xi_ref[...]
    zu = (jnp.dot(xu, wdiu_ref[...], preferred_element_type=jnp.float32)
          + jnp.dot(xu, wduu_ref[...], preferred_element_type=jnp.float32)
          + biu_ref[...] + buu_ref[...])
    zi = (jnp.dot(xi, wdui_ref[...], preferred_element_type=jnp.float32)
          + bui_ref[...])
    au = jnp.concatenate(
        [aiu_ref[c].astype(jnp.float32) * riu
         + auu_ref[c].astype(jnp.float32) * ruu for c in range(NCH)], axis=1)
    ai = jnp.concatenate(
        [aui_ref[c].astype(jnp.float32) * rui for c in range(NCH)], axis=1)
    ou_ref[...] = au + zu
    oi_ref[...] = ai + zi


def _combine(aui, aiu, auu, dgui, dgiu, dguu, x_user, x_item,
             wdui, wdiu, wduu, bui, biu, buu):
    aspec = pl.BlockSpec((NCH, BM, CW), lambda r: (0, r, 0))
    dspec = pl.BlockSpec((2, BM, 1), lambda r: (0, r, 0))
    xspec = pl.BlockSpec((BM, D), lambda r: (r, 0))
    wspec = pl.BlockSpec((D, D), lambda r: (0, 0))
    bspec = pl.BlockSpec((1, D), lambda r: (0, 0))
    ospec = pl.BlockSpec((BM, D), lambda r: (r, 0))
    oshape = jax.ShapeDtypeStruct((N, D), jnp.float32)
    return pl.pallas_call(
        _combine_body,
        grid=(N // BM,),
        in_specs=[aspec, aspec, aspec, dspec, dspec, dspec,
                  xspec, xspec, wspec, wspec, wspec, bspec, bspec, bspec],
        out_specs=[ospec, ospec],
        out_shape=[oshape, oshape],
    )(aui, aiu, auu, dgui, dgiu, dguu, x_user, x_item,
      wdui, wdiu, wduu, bui, biu, buu)


def _sc_body(yui, yiu, yuu, sui, siu, suu, dui, diu, duu,
             aui, aiu, auu, dgui, dgiu, dguu,
             acc, sdeg, rows, sidx, didx, degstage, zdeg, ones,
             gsems, ssems, dsems, wsem):
    cid = lax.axis_index("c")
    sid = lax.axis_index("s")
    z16 = jnp.zeros((16,), jnp.float32)
    o16 = jnp.ones((16,), jnp.float32)

    def zd(i, c):
        zdeg[pl.ds(i * 16, 16)] = z16
        return c
    lax.fori_loop(0, zdeg.shape[0] // 16, zd, 0)
    for j in range(MICRO // 16):
        ones[pl.ds(j * 16, 16)] = o16

    rowsA = rows.at[pl.ds(0, MICRO)]
    rowsB = rows.at[pl.ds(MICRO, MICRO)]

    def run_type(yref, sref, dref, aggref, degref):
        if True:
            ch = cid
            plsc.subcore_barrier()

            zb16 = jnp.zeros((32,), jnp.bfloat16)

            def zr(i, c):
                rows[i, pl.ds(0, 32)] = zb16
                rows[i, pl.ds(32, 32)] = zb16
                return c
            lax.fori_loop(0, MICRO, zr, 0)
            zb = sid * WPT
            zcps = [pltpu.async_copy(
                        rowsA, acc.at[pl.ds(zb + MICRO * k, MICRO)], wsem)
                    for k in range(WPT // MICRO)]
            zcps.append(pltpu.async_copy(
                rows.at[pl.ds(0, WPT % MICRO)],
                acc.at[pl.ds(zb + WPT - WPT % MICRO, WPT % MICRO)], wsem))

            def zdg(k, c):
                pltpu.sync_copy(zdeg.at[pl.ds(0, 136)],
                                sdeg.at[pl.ds(sid * DPT + 136 * k, 136)])
                return c
            lax.fori_loop(0, DPT // 136, zdg, 0)
            for cp in zcps:
                cp.wait()
            plsc.subcore_barrier()

            dbase = sid * TROWS

            def quarter(q, cq):
                pltpu.sync_copy(sref.at[pl.ds(dbase + q * QROWS, QROWS)], sidx)
                pltpu.sync_copy(dref.at[pl.ds(dbase + q * QROWS, QROWS)], didx)
                choff16 = jnp.broadcast_to(ch * N, (16,)).astype(jnp.int32)

                def addoff(i, c):
                    for h in range(MICRO // 16):
                        sidx[i, pl.ds(16 * h, 16)] = (
                            sidx[i, pl.ds(16 * h, 16)] + choff16)
                    return c
                lax.fori_loop(0, QROWS, addoff, 0)

                def slot(j):
                    return rows.at[pl.ds(MICRO * j, MICRO)]

                def fire_g(t, j):
                    pltpu.async_copy(yref.at[sidx.at[t]], slot(j), gsems[j])

                def wait_g(t, j):
                    pltpu.make_async_copy(yref.at[sidx.at[t]], slot(j),
                                          gsems[j]).wait()

                def fire_s(t, j):
                    pltpu.async_copy(slot(j), acc.at[didx.at[t]], ssems[j],
                                     add=True)

                def wait_s(t, j):
                    pltpu.make_async_copy(slot(j), acc.at[didx.at[t]],
                                          ssems[j]).wait()

                fire_g(0, 0)
                fire_g(1, 1)

                def step(u, cu):
                    t0 = 4 * u
                    for j in range(4):
                        t = t0 + j

                        @pl.when(t >= 2)
                        def _(t=t, j=j):
                            wait_s(t - 2, (j + 2) % 4)

                        @pl.when(t + 2 < QROWS)
                        def _(t=t, j=j):
                            fire_g(t + 2, (j + 2) % 4)
                        wait_g(t, j)
                        fire_s(t, j)
                    return cu
                lax.fori_loop(0, QROWS // 4, step, 0)
                wait_s(QROWS - 2, 2)
                wait_s(QROWS - 1, 3)
                return cq
            lax.fori_loop(0, TROWS // QROWS, quarter, 0)

            dstart = dbase + cid * (TROWS // 2)

            def degblk(b, c):
                pltpu.sync_copy(dref.at[pl.ds(dstart + b * QROWS, QROWS)],
                                didx)

                def dstep(rr, c2):
                    for j in range(4):
                        r = 4 * rr + j

                        @pl.when(rr > 0)
                        def _(r=r, j=j):
                            pltpu.make_async_copy(
                                ones, sdeg.at[didx.at[r - 4]], dsems[j]).wait()
                        pltpu.async_copy(ones, sdeg.at[didx.at[r]], dsems[j],
                                         add=True)
                    return c2
                lax.fori_loop(0, QROWS // 4, dstep, 0)
                for j in range(4):
                    pltpu.make_async_copy(
                        ones, sdeg.at[didx.at[QROWS - 4 + j]], dsems[j]).wait()
                return c
            lax.fori_loop(0, TROWS // 2 // QROWS, degblk, 0)
            plsc.subcore_barrier()

            wb = sid * WPT

            def wp(k, c):
                @pl.when(k > 0)
                def _():
                    pltpu.make_async_copy(
                        rowsA, aggref.at[pl.ds(0, MICRO)], wsem).wait()
                    pltpu.make_async_copy(
                        rowsB, aggref.at[pl.ds(0, MICRO)], wsem).wait()
                pltpu.sync_copy(acc.at[pl.ds(wb + MICRO * 2 * k, MICRO)],
                                rowsA)
                pltpu.async_copy(
                    rowsA,
                    aggref.at[pl.ds(ch * NPAD + wb + MICRO * 2 * k, MICRO)],
                    wsem)
                pltpu.sync_copy(acc.at[pl.ds(wb + MICRO * (2 * k + 1), MICRO)],
                                rowsB)
                pltpu.async_copy(
                    rowsB,
                    aggref.at[pl.ds(ch * NPAD + wb + MICRO * (2 * k + 1),
                                    MICRO)],
                    wsem)
                return c
            lax.fori_loop(0, WPT // (2 * MICRO), wp, 0)
            pltpu.make_async_copy(rowsA, aggref.at[pl.ds(0, MICRO)],
                                  wsem).wait()
            pltpu.make_async_copy(rowsB, aggref.at[pl.ds(0, MICRO)],
                                  wsem).wait()
            pltpu.sync_copy(acc.at[pl.ds(wb + WPT - WPT % MICRO, WPT % MICRO)],
                            rows.at[pl.ds(0, WPT % MICRO)])
            pltpu.sync_copy(rows.at[pl.ds(0, WPT % MICRO)],
                            aggref.at[pl.ds(ch * NPAD + wb + WPT - WPT % MICRO,
                                            WPT % MICRO)])

            def wdg(k, c):
                pltpu.sync_copy(sdeg.at[pl.ds(sid * DPT + 136 * k, 136)],
                                degstage.at[pl.ds(0, 136)])
                pltpu.sync_copy(
                    degstage.at[pl.ds(0, 136)],
                    degref.at[pl.ds(cid * DEG_ROWS + sid * DPT + 136 * k,
                                    136)])
                return c
            lax.fori_loop(0, DPT // 136, wdg, 0)

    run_type(yui, sui, dui, aui, dgui)
    run_type(yiu, siu, diu, aiu, dgiu)
    run_type(yuu, suu, duu, auu, dguu)


def _sc_gather_scatter(yui, yiu, yuu, sui, siu, suu, dui, diu, duu):
    mesh = plsc.VectorSubcoreMesh(core_axis_name="c", subcore_axis_name="s",
                                  num_cores=2, num_subcores=NSUB)
    agg = jax.ShapeDtypeStruct((NCH * NPAD, CW), jnp.bfloat16)
    deg = jax.ShapeDtypeStruct((2 * DEG_ROWS,), jnp.float32)
    f = pl.kernel(
        _sc_body,
        out_type=[agg, agg, agg, deg, deg, deg],
        mesh=mesh,
        compiler_params=pltpu.CompilerParams(use_tc_tiling_on_sc=False),
        scratch_types=[
            pltpu.VMEM_SHARED((ACC_ROWS, CW), jnp.bfloat16),
            pltpu.VMEM_SHARED((DEG_ROWS,), jnp.float32),
            pltpu.VMEM((4 * MICRO, CW), jnp.bfloat16),
            pltpu.VMEM((QROWS, MICRO), jnp.int32),
            pltpu.VMEM((QROWS, MICRO), jnp.int32),
            pltpu.VMEM((144,), jnp.float32),
            pltpu.VMEM((144,), jnp.float32),
            pltpu.VMEM((MICRO,), jnp.float32),
            [pltpu.SemaphoreType.DMA] * 4,
            [pltpu.SemaphoreType.DMA] * 4,
            [pltpu.SemaphoreType.DMA] * 4,
            pltpu.SemaphoreType.DMA,
        ],
    )
    return f(yui, yiu, yuu, sui, siu, suu, dui, diu, duu)


def kernel(x_user, x_item, ei_user_item, ei_item_user, ei_user_user,
           W_src_ui, W_dst_ui, b_ui,
           W_src_iu, W_dst_iu, b_iu,
           W_src_uu, W_dst_uu, b_uu):
    yui, yiu, yuu = _y_matmuls(x_user, x_item, W_src_ui, W_src_iu, W_src_uu)

    def prep_src(ei):
        return jnp.concatenate(
            [ei[0], jnp.zeros((EPAD - E,), jnp.int32)]).reshape(IDROWS, MICRO)

    def prep_dst(ei):
        return jnp.concatenate(
            [ei[1], jnp.full((EPAD - E,), N, jnp.int32)]).reshape(IDROWS, MICRO)

    sui, dui = prep_src(ei_user_item), prep_dst(ei_user_item)
    siu, diu = prep_src(ei_item_user), prep_dst(ei_item_user)
    suu, duu = prep_src(ei_user_user), prep_dst(ei_user_user)

    aui, aiu, auu, dgui, dgiu, dguu = _sc_gather_scatter(
        yui.reshape(NCH * N, CW), yiu.reshape(NCH * N, CW),
        yuu.reshape(NCH * N, CW), sui, siu, suu, dui, diu, duu)

    out_user, out_item = _combine(
        aui.reshape(NCH, NPAD, CW), aiu.reshape(NCH, NPAD, CW),
        auu.reshape(NCH, NPAD, CW),
        dgui.reshape(2, DEG_ROWS, 1), dgiu.reshape(2, DEG_ROWS, 1),
        dguu.reshape(2, DEG_ROWS, 1),
        x_user, x_item, W_dst_ui, W_dst_iu, W_dst_uu,
        b_ui.reshape(1, D), b_iu.reshape(1, D), b_uu.reshape(1, D))
    return out_user, out_item


# speedup vs baseline: 1.1100x; 1.1100x over previous
import functools

import jax
import jax.numpy as jnp
from jax import lax
from jax.experimental import pallas as pl
from jax.experimental.pallas import tpu as pltpu
from jax.experimental.pallas import tpu_sc as plsc

N = 50000
D = 128
E = 500000
CW = 64
NCH = D // CW
EPAD = 524288
NSUB = 16
EPT = EPAD // NSUB
MICRO = 128
QROWS = 32
IDROWS = EPAD // MICRO
TROWS = EPT // MICRO
NPAD = 50048
ACC_ROWS = NPAD
WPT = NPAD // NSUB
DEG_ROWS = NPAD
DPT = DEG_ROWS // NSUB
BM = 2000


def _y_body(xu_ref, xi_ref, wui_ref, wiu_ref, wuu_ref, yui_ref, yiu_ref, yuu_ref):
    xu = xu_ref[...]
    xi = xi_ref[...]
    yui = jnp.dot(xu, wui_ref[...], preferred_element_type=jnp.float32)
    yiu = jnp.dot(xi, wiu_ref[...], preferred_element_type=jnp.float32)
    yuu = jnp.dot(xu, wuu_ref[...], preferred_element_type=jnp.float32)
    for c in range(NCH):
        yui_ref[c] = yui[:, c * CW:(c + 1) * CW].astype(jnp.bfloat16)
        yiu_ref[c] = yiu[:, c * CW:(c + 1) * CW].astype(jnp.bfloat16)
        yuu_ref[c] = yuu[:, c * CW:(c + 1) * CW].astype(jnp.bfloat16)


def _y_matmuls(x_user, x_item, w_ui, w_iu, w_uu):
    xspec = pl.BlockSpec((BM, D), lambda r: (r, 0))
    wspec = pl.BlockSpec((D, D), lambda r: (0, 0))
    yspec = pl.BlockSpec((NCH, BM, CW), lambda r: (0, r, 0))
    yshape = jax.ShapeDtypeStruct((NCH, N, CW), jnp.bfloat16)
    return pl.pallas_call(
        _y_body,
        grid=(N // BM,),
        in_specs=[xspec, xspec, wspec, wspec, wspec],
        out_specs=[yspec, yspec, yspec],
        out_shape=[yshape, yshape, yshape],
    )(x_user, x_item, w_ui, w_iu, w_uu)


def _combine_body(aui_ref, aiu_ref, auu_ref, dui_ref, diu_ref, duu_ref,
                  xu_ref, xi_ref, wdui_ref, wdiu_ref, wduu_ref,
                  bui_ref, biu_ref, buu_ref, ou_ref, oi_ref):
    rui = 1.0 / jnp.maximum(dui_ref[...], 1.0)
    riu = 1.0 / jnp.maximum(diu_ref[...], 1.0)
    ruu = 1.0 / jnp.maximum(duu_ref[...], 1.0)
    xu = xu_ref[...]
    xi = xi_ref[...]
    zu = (jnp.dot(xu, wdiu_ref[...], preferred_element_type=jnp.float32)
          + jnp.dot(xu, wduu_ref[...], preferred_element_type=jnp.float32)
          + biu_ref[...] + buu_ref[...])
    zi = (jnp.dot(xi, wdui_ref[...], preferred_element_type=jnp.float32)
          + bui_ref[...])
    au = jnp.concatenate(
        [aiu_ref[c].astype(jnp.float32) * riu
         + auu_ref[c].astype(jnp.float32) * ruu for c in range(NCH)], axis=1)
    ai = jnp.concatenate(
        [aui_ref[c].astype(jnp.float32) * rui for c in range(NCH)], axis=1)
    ou_ref[...] = au + zu
    oi_ref[...] = ai + zi


def _combine(aui, aiu, auu, dgui, dgiu, dguu, x_user, x_item,
             wdui, wdiu, wduu, bui, biu, buu):
    aspec = pl.BlockSpec((NCH, BM, CW), lambda r: (0, r, 0))
    dspec = pl.BlockSpec((BM, 1), lambda r: (r, 0))
    xspec = pl.BlockSpec((BM, D), lambda r: (r, 0))
    wspec = pl.BlockSpec((D, D), lambda r: (0, 0))
    bspec = pl.BlockSpec((1, D), lambda r: (0, 0))
    ospec = pl.BlockSpec((BM, D), lambda r: (r, 0))
    oshape = jax.ShapeDtypeStruct((N, D), jnp.float32)
    return pl.pallas_call(
        _combine_body,
        grid=(N // BM,),
        in_specs=[aspec, aspec, aspec, dspec, dspec, dspec,
                  xspec, xspec, wspec, wspec, wspec, bspec, bspec, bspec],
        out_specs=[ospec, ospec],
        out_shape=[oshape, oshape],
    )(aui, aiu, auu, dgui, dgiu, dguu, x_user, x_item,
      wdui, wdiu, wduu, bui, biu, buu)


def _sc_body(yui, yiu, yuu, sui, siu, suu, dui, diu, duu,
             aui, aiu, auu, dgui, dgiu, dguu,
             acc, sdeg, rows, sidx, didx, degstage, zdeg, ones,
             gsems, ssems, dsems, wsem):
    cid = lax.axis_index("c")
    sid = lax.axis_index("s")
    z16 = jnp.zeros((16,), jnp.float32)
    o16 = jnp.ones((16,), jnp.float32)

    def zd(i, c):
        zdeg[pl.ds(i * 16, 16)] = z16
        return c
    lax.fori_loop(0, zdeg.shape[0] // 16, zd, 0)
    for j in range(MICRO // 16):
        ones[pl.ds(j * 16, 16)] = o16

    rowsA = rows.at[pl.ds(0, MICRO)]
    rowsB = rows.at[pl.ds(MICRO, MICRO)]

    def run_type(yref, sref, dref, aggref, degref, deg_core):
        if True:
            ch = cid
            do_deg = cid == deg_core
            plsc.subcore_barrier()

            zb16 = jnp.zeros((32,), jnp.bfloat16)

            def zr(i, c):
                rows[i, pl.ds(0, 32)] = zb16
                rows[i, pl.ds(32, 32)] = zb16
                return c
            lax.fori_loop(0, MICRO, zr, 0)
            zb = sid * WPT
            zcps = [pltpu.async_copy(
                        rowsA, acc.at[pl.ds(zb + MICRO * k, MICRO)], wsem)
                    for k in range(WPT // MICRO)]
            zcps.append(pltpu.async_copy(
                rows.at[pl.ds(0, WPT % MICRO)],
                acc.at[pl.ds(zb + WPT - WPT % MICRO, WPT % MICRO)], wsem))

            @pl.when(do_deg)
            def _():
                def zdg(k, c):
                    pltpu.sync_copy(zdeg.at[pl.ds(0, 136)],
                                    sdeg.at[pl.ds(sid * DPT + 136 * k, 136)])
                    return c
                lax.fori_loop(0, DPT // 136, zdg, 0)
            for cp in zcps:
                cp.wait()
            plsc.subcore_barrier()

            dbase = sid * TROWS

            def quarter(q, cq):
                pltpu.sync_copy(sref.at[pl.ds(dbase + q * QROWS, QROWS)], sidx)
                pltpu.sync_copy(dref.at[pl.ds(dbase + q * QROWS, QROWS)], didx)
                choff16 = jnp.broadcast_to(ch * N, (16,)).astype(jnp.int32)

                def addoff(i, c):
                    for h in range(MICRO // 16):
                        sidx[i, pl.ds(16 * h, 16)] = (
                            sidx[i, pl.ds(16 * h, 16)] + choff16)
                    return c
                lax.fori_loop(0, QROWS, addoff, 0)

                def slot(j):
                    return rows.at[pl.ds(MICRO * j, MICRO)]

                def fire_g(t, j):
                    pltpu.async_copy(yref.at[sidx.at[t]], slot(j), gsems[j])

                def wait_g(t, j):
                    pltpu.make_async_copy(yref.at[sidx.at[t]], slot(j),
                                          gsems[j]).wait()

                def fire_s(t, j):
                    pltpu.async_copy(slot(j), acc.at[didx.at[t]], ssems[j],
                                     add=True)

                    @pl.when(do_deg)
                    def _():
                        pltpu.async_copy(ones, sdeg.at[didx.at[t]], dsems[j],
                                         add=True)

                def wait_s(t, j):
                    pltpu.make_async_copy(slot(j), acc.at[didx.at[t]],
                                          ssems[j]).wait()

                    @pl.when(do_deg)
                    def _():
                        pltpu.make_async_copy(ones, sdeg.at[didx.at[t]],
                                              dsems[j]).wait()

                fire_g(0, 0)
                fire_g(1, 1)
                fire_g(2, 2)

                def step(u, cu):
                    t0 = 4 * u
                    for j in range(4):
                        t = t0 + j

                        @pl.when(t >= 1)
                        def _(t=t, j=j):
                            wait_s(t - 1, (j + 3) % 4)

                        @pl.when(t + 3 < QROWS)
                        def _(t=t, j=j):
                            fire_g(t + 3, (j + 3) % 4)
                        wait_g(t, j)
                        fire_s(t, j)
                    return cu
                lax.fori_loop(0, QROWS // 4, step, 0)
                wait_s(QROWS - 1, 3)
                return cq
            lax.fori_loop(0, TROWS // QROWS, quarter, 0)
            plsc.subcore_barrier()

            wb = sid * WPT

            def wp(k, c):
                @pl.when(k > 0)
                def _():
                    pltpu.make_async_copy(
                        rowsA, aggref.at[pl.ds(0, MICRO)], wsem).wait()
                    pltpu.make_async_copy(
                        rowsB, aggref.at[pl.ds(0, MICRO)], wsem).wait()
                pltpu.sync_copy(acc.at[pl.ds(wb + MICRO * 2 * k, MICRO)],
                                rowsA)
                pltpu.async_copy(
                    rowsA,
                    aggref.at[pl.ds(ch * NPAD + wb + MICRO * 2 * k, MICRO)],
                    wsem)
                pltpu.sync_copy(acc.at[pl.ds(wb + MICRO * (2 * k + 1), MICRO)],
                                rowsB)
                pltpu.async_copy(
                    rowsB,
                    aggref.at[pl.ds(ch * NPAD + wb + MICRO * (2 * k + 1),
                                    MICRO)],
                    wsem)
                return c
            lax.fori_loop(0, WPT // (2 * MICRO), wp, 0)
            pltpu.make_async_copy(rowsA, aggref.at[pl.ds(0, MICRO)],
                                  wsem).wait()
            pltpu.make_async_copy(rowsB, aggref.at[pl.ds(0, MICRO)],
                                  wsem).wait()
            pltpu.sync_copy(acc.at[pl.ds(wb + WPT - WPT % MICRO, WPT % MICRO)],
                            rows.at[pl.ds(0, WPT % MICRO)])
            pltpu.sync_copy(rows.at[pl.ds(0, WPT % MICRO)],
                            aggref.at[pl.ds(ch * NPAD + wb + WPT - WPT % MICRO,
                                            WPT % MICRO)])

            @pl.when(do_deg)
            def _():
                def wdg(k, c):
                    pltpu.sync_copy(sdeg.at[pl.ds(sid * DPT + 136 * k, 136)],
                                    degstage.at[pl.ds(0, 136)])
                    pltpu.sync_copy(degstage.at[pl.ds(0, 136)],
                                    degref.at[pl.ds(sid * DPT + 136 * k, 136)])
                    return c
                lax.fori_loop(0, DPT // 136, wdg, 0)

    run_type(yui, sui, dui, aui, dgui, 0)
    run_type(yiu, siu, diu, aiu, dgiu, 1)
    run_type(yuu, suu, duu, auu, dguu, 1)


def _sc_gather_scatter(yui, yiu, yuu, sui, siu, suu, dui, diu, duu):
    mesh = plsc.VectorSubcoreMesh(core_axis_name="c", subcore_axis_name="s",
                                  num_cores=2, num_subcores=NSUB)
    agg = jax.ShapeDtypeStruct((NCH * NPAD, CW), jnp.bfloat16)
    deg = jax.ShapeDtypeStruct((DEG_ROWS,), jnp.float32)
    f = pl.kernel(
        _sc_body,
        out_type=[agg, agg, agg, deg, deg, deg],
        mesh=mesh,
        compiler_params=pltpu.CompilerParams(use_tc_tiling_on_sc=False),
        scratch_types=[
            pltpu.VMEM_SHARED((ACC_ROWS, CW), jnp.bfloat16),
            pltpu.VMEM_SHARED((DEG_ROWS,), jnp.float32),
            pltpu.VMEM((4 * MICRO, CW), jnp.bfloat16),
            pltpu.VMEM((QROWS, MICRO), jnp.int32),
            pltpu.VMEM((QROWS, MICRO), jnp.int32),
            pltpu.VMEM((144,), jnp.float32),
            pltpu.VMEM((144,), jnp.float32),
            pltpu.VMEM((MICRO,), jnp.float32),
            [pltpu.SemaphoreType.DMA] * 4,
            [pltpu.SemaphoreType.DMA] * 4,
            [pltpu.SemaphoreType.DMA] * 4,
            pltpu.SemaphoreType.DMA,
        ],
    )
    return f(yui, yiu, yuu, sui, siu, suu, dui, diu, duu)


def kernel(x_user, x_item, ei_user_item, ei_item_user, ei_user_user,
           W_src_ui, W_dst_ui, b_ui,
           W_src_iu, W_dst_iu, b_iu,
           W_src_uu, W_dst_uu, b_uu):
    yui, yiu, yuu = _y_matmuls(x_user, x_item, W_src_ui, W_src_iu, W_src_uu)

    def prep_src(ei):
        return jnp.concatenate(
            [ei[0], jnp.zeros((EPAD - E,), jnp.int32)]).reshape(IDROWS, MICRO)

    def prep_dst(ei):
        return jnp.concatenate(
            [ei[1], jnp.full((EPAD - E,), N, jnp.int32)]).reshape(IDROWS, MICRO)

    sui, dui = prep_src(ei_user_item), prep_dst(ei_user_item)
    siu, diu = prep_src(ei_item_user), prep_dst(ei_item_user)
    suu, duu = prep_src(ei_user_user), prep_dst(ei_user_user)

    aui, aiu, auu, dgui, dgiu, dguu = _sc_gather_scatter(
        yui.reshape(NCH * N, CW), yiu.reshape(NCH * N, CW),
        yuu.reshape(NCH * N, CW), sui, siu, suu, dui, diu, duu)

    out_user, out_item = _combine(
        aui.reshape(NCH, NPAD, CW), aiu.reshape(NCH, NPAD, CW),
        auu.reshape(NCH, NPAD, CW),
        dgui.reshape(DEG_ROWS, 1), dgiu.reshape(DEG_ROWS, 1),
        dguu.reshape(DEG_ROWS, 1),
        x_user, x_item, W_dst_ui, W_dst_iu, W_dst_uu,
        b_ui.reshape(1, D), b_iu.reshape(1, D), b_uu.reshape(1, D))
    return out_user, out_item


# speedup vs baseline: 1.5708x; 1.4151x over previous
import functools

import jax
import jax.numpy as jnp
from jax import lax
from jax.experimental import pallas as pl
from jax.experimental.pallas import tpu as pltpu
from jax.experimental.pallas import tpu_sc as plsc

N = 50000
D = 128
E = 500000
CW = 32
NCH = D // CW
EPAD = 524288
NSUB = 16
EPT = EPAD // NSUB
MICRO = 128
QROWS = 32
IDROWS = EPAD // MICRO
TROWS = EPT // MICRO
NPAD = 50048
ACC_ROWS = NPAD
WPT = NPAD // NSUB
DEG_ROWS = NPAD
DPT = DEG_ROWS // NSUB
BM = 2000


def _y_body(xu_ref, xi_ref, wui_ref, wiu_ref, wuu_ref, yui_ref, yiu_ref, yuu_ref):
    xu = xu_ref[...]
    xi = xi_ref[...]
    yui = jnp.dot(xu, wui_ref[...], preferred_element_type=jnp.float32)
    yiu = jnp.dot(xi, wiu_ref[...], preferred_element_type=jnp.float32)
    yuu = jnp.dot(xu, wuu_ref[...], preferred_element_type=jnp.float32)
    for c in range(NCH):
        yui_ref[c] = yui[:, c * CW:(c + 1) * CW].astype(jnp.bfloat16)
        yiu_ref[c] = yiu[:, c * CW:(c + 1) * CW].astype(jnp.bfloat16)
        yuu_ref[c] = yuu[:, c * CW:(c + 1) * CW].astype(jnp.bfloat16)


def _y_matmuls(x_user, x_item, w_ui, w_iu, w_uu):
    xspec = pl.BlockSpec((BM, D), lambda r: (r, 0))
    wspec = pl.BlockSpec((D, D), lambda r: (0, 0))
    yspec = pl.BlockSpec((NCH, BM, CW), lambda r: (0, r, 0))
    yshape = jax.ShapeDtypeStruct((NCH, NPAD, CW), jnp.bfloat16)
    return pl.pallas_call(
        _y_body,
        grid=(N // BM,),
        in_specs=[xspec, xspec, wspec, wspec, wspec],
        out_specs=[yspec, yspec, yspec],
        out_shape=[yshape, yshape, yshape],
    )(x_user, x_item, w_ui, w_iu, w_uu)


def _combine_body(aui_ref, aiu_ref, auu_ref, dui_ref, diu_ref, duu_ref,
                  xu_ref, xi_ref, wdui_ref, wdiu_ref, wduu_ref,
                  bui_ref, biu_ref, buu_ref, ou_ref, oi_ref):
    rui = 1.0 / jnp.maximum(dui_ref[...], 1.0)
    riu = 1.0 / jnp.maximum(diu_ref[...], 1.0)
    ruu = 1.0 / jnp.maximum(duu_ref[...], 1.0)
    xu = xu_ref[...]
    xi = xi_ref[...]
    zu = (jnp.dot(xu, wdiu_ref[...], preferred_element_type=jnp.float32)
          + jnp.dot(xu, wduu_ref[...], preferred_element_type=jnp.float32)
          + biu_ref[...] + buu_ref[...])
    zi = (jnp.dot(xi, wdui_ref[...], preferred_element_type=jnp.float32)
          + bui_ref[...])
    au = jnp.concatenate(
        [aiu_ref[c].astype(jnp.float32) * riu
         + auu_ref[c].astype(jnp.float32) * ruu for c in range(NCH)], axis=1)
    ai = jnp.concatenate(
        [aui_ref[c].astype(jnp.float32) * rui for c in range(NCH)], axis=1)
    ou_ref[...] = au + zu
    oi_ref[...] = ai + zi


def _combine(aui, aiu, auu, dgui, dgiu, dguu, x_user, x_item,
             wdui, wdiu, wduu, bui, biu, buu):
    aspec = pl.BlockSpec((NCH, BM, CW), lambda r: (0, r, 0))
    dspec = pl.BlockSpec((BM, 1), lambda r: (r, 0))
    xspec = pl.BlockSpec((BM, D), lambda r: (r, 0))
    wspec = pl.BlockSpec((D, D), lambda r: (0, 0))
    bspec = pl.BlockSpec((1, D), lambda r: (0, 0))
    ospec = pl.BlockSpec((BM, D), lambda r: (r, 0))
    oshape = jax.ShapeDtypeStruct((N, D), jnp.float32)
    return pl.pallas_call(
        _combine_body,
        grid=(N // BM,),
        in_specs=[aspec, aspec, aspec, dspec, dspec, dspec,
                  xspec, xspec, wspec, wspec, wspec, bspec, bspec, bspec],
        out_specs=[ospec, ospec],
        out_shape=[oshape, oshape],
    )(aui, aiu, auu, dgui, dgiu, dguu, x_user, x_item,
      wdui, wdiu, wduu, bui, biu, buu)


def _sc_body(yui, yiu, yuu, sui, siu, suu, dui, diu, duu,
             aui, aiu, auu, dgui, dgiu, dguu,
             acc, sdeg, tab, rows, sidx, didx, degstage, zdeg, ones,
             gsems, ssems, dsems, wsem):
    cid = lax.axis_index("c")
    sid = lax.axis_index("s")
    z16 = jnp.zeros((16,), jnp.float32)
    o16 = jnp.ones((16,), jnp.float32)

    def zd(i, c):
        zdeg[pl.ds(i * 16, 16)] = z16
        return c
    lax.fori_loop(0, zdeg.shape[0] // 16, zd, 0)
    for j in range(MICRO // 16):
        ones[pl.ds(j * 16, 16)] = o16

    rowsA = rows.at[pl.ds(0, MICRO)]
    rowsB = rows.at[pl.ds(MICRO, MICRO)]

    def run_type(yref, sref, dref, aggref, degref, deg_core):
        def pass_body(p, carry):
            ch = 2 * cid + p
            do_deg = jnp.logical_and(cid == deg_core, p == 0)
            plsc.subcore_barrier()

            sbase = ch * NPAD + sid * WPT
            tbase = sid * WPT

            def stg(k, c):
                pltpu.sync_copy(yref.at[pl.ds(sbase + 512 * k, 512)], rows)
                pltpu.sync_copy(rows, tab.at[pl.ds(tbase + 512 * k, 512)])
                return c
            lax.fori_loop(0, WPT // 512, stg, 0)
            pltpu.sync_copy(yref.at[pl.ds(sbase + WPT - WPT % 512, WPT % 512)],
                            rows.at[pl.ds(0, WPT % 512)])
            pltpu.sync_copy(rows.at[pl.ds(0, WPT % 512)],
                            tab.at[pl.ds(tbase + WPT - WPT % 512, WPT % 512)])

            zb16 = jnp.zeros((32,), jnp.bfloat16)

            def zr(i, c):
                rows[i, pl.ds(0, 32)] = zb16
                return c
            lax.fori_loop(0, MICRO, zr, 0)
            zb = sid * WPT
            zcps = [pltpu.async_copy(
                        rowsA, acc.at[pl.ds(zb + MICRO * k, MICRO)], wsem)
                    for k in range(WPT // MICRO)]
            zcps.append(pltpu.async_copy(
                rows.at[pl.ds(0, WPT % MICRO)],
                acc.at[pl.ds(zb + WPT - WPT % MICRO, WPT % MICRO)], wsem))

            @pl.when(do_deg)
            def _():
                def zdg(k, c):
                    pltpu.sync_copy(zdeg.at[pl.ds(0, 136)],
                                    sdeg.at[pl.ds(sid * DPT + 136 * k, 136)])
                    return c
                lax.fori_loop(0, DPT // 136, zdg, 0)
            for cp in zcps:
                cp.wait()
            plsc.subcore_barrier()

            dbase = sid * TROWS

            def quarter(q, cq):
                pltpu.sync_copy(sref.at[pl.ds(dbase + q * QROWS, QROWS)], sidx)
                pltpu.sync_copy(dref.at[pl.ds(dbase + q * QROWS, QROWS)], didx)

                def slot(j):
                    return rows.at[pl.ds(MICRO * j, MICRO)]

                def fire_g(t, j):
                    pltpu.async_copy(tab.at[sidx.at[t]], slot(j), gsems[j])

                def wait_g(t, j):
                    pltpu.make_async_copy(tab.at[sidx.at[t]], slot(j),
                                          gsems[j]).wait()

                def fire_s(t, j):
                    pltpu.async_copy(slot(j), acc.at[didx.at[t]], ssems[j],
                                     add=True)

                    @pl.when(do_deg)
                    def _():
                        pltpu.async_copy(ones, sdeg.at[didx.at[t]], dsems[j],
                                         add=True)

                def wait_s(t, j):
                    pltpu.make_async_copy(slot(j), acc.at[didx.at[t]],
                                          ssems[j]).wait()

                    @pl.when(do_deg)
                    def _():
                        pltpu.make_async_copy(ones, sdeg.at[didx.at[t]],
                                              dsems[j]).wait()

                fire_g(0, 0)
                fire_g(1, 1)
                fire_g(2, 2)

                def step(u, cu):
                    t0 = 4 * u
                    for j in range(4):
                        t = t0 + j

                        @pl.when(t >= 1)
                        def _(t=t, j=j):
                            wait_s(t - 1, (j + 3) % 4)

                        @pl.when(t + 3 < QROWS)
                        def _(t=t, j=j):
                            fire_g(t + 3, (j + 3) % 4)
                        wait_g(t, j)
                        fire_s(t, j)
                    return cu
                lax.fori_loop(0, QROWS // 4, step, 0)
                wait_s(QROWS - 1, 3)
                return cq
            lax.fori_loop(0, TROWS // QROWS, quarter, 0)
            plsc.subcore_barrier()

            wb = sid * WPT

            def wp(k, c):
                @pl.when(k > 0)
                def _():
                    pltpu.make_async_copy(
                        rowsA, aggref.at[pl.ds(0, MICRO)], wsem).wait()
                    pltpu.make_async_copy(
                        rowsB, aggref.at[pl.ds(0, MICRO)], wsem).wait()
                pltpu.sync_copy(acc.at[pl.ds(wb + MICRO * 2 * k, MICRO)],
                                rowsA)
                pltpu.async_copy(
                    rowsA,
                    aggref.at[pl.ds(ch * NPAD + wb + MICRO * 2 * k, MICRO)],
                    wsem)
                pltpu.sync_copy(acc.at[pl.ds(wb + MICRO * (2 * k + 1), MICRO)],
                                rowsB)
                pltpu.async_copy(
                    rowsB,
                    aggref.at[pl.ds(ch * NPAD + wb + MICRO * (2 * k + 1),
                                    MICRO)],
                    wsem)
                return c
            lax.fori_loop(0, WPT // (2 * MICRO), wp, 0)
            pltpu.make_async_copy(rowsA, aggref.at[pl.ds(0, MICRO)],
                                  wsem).wait()
            pltpu.make_async_copy(rowsB, aggref.at[pl.ds(0, MICRO)],
                                  wsem).wait()
            pltpu.sync_copy(acc.at[pl.ds(wb + WPT - WPT % MICRO, WPT % MICRO)],
                            rows.at[pl.ds(0, WPT % MICRO)])
            pltpu.sync_copy(rows.at[pl.ds(0, WPT % MICRO)],
                            aggref.at[pl.ds(ch * NPAD + wb + WPT - WPT % MICRO,
                                            WPT % MICRO)])

            @pl.when(do_deg)
            def _():
                def wdg(k, c):
                    pltpu.sync_copy(sdeg.at[pl.ds(sid * DPT + 136 * k, 136)],
                                    degstage.at[pl.ds(0, 136)])
                    pltpu.sync_copy(degstage.at[pl.ds(0, 136)],
                                    degref.at[pl.ds(sid * DPT + 136 * k, 136)])
                    return c
                lax.fori_loop(0, DPT // 136, wdg, 0)
            return carry
        lax.fori_loop(0, 2, pass_body, 0)

    run_type(yui, sui, dui, aui, dgui, 0)
    run_type(yiu, siu, diu, aiu, dgiu, 1)
    run_type(yuu, suu, duu, auu, dguu, 1)


def _sc_gather_scatter(yui, yiu, yuu, sui, siu, suu, dui, diu, duu):
    mesh = plsc.VectorSubcoreMesh(core_axis_name="c", subcore_axis_name="s",
                                  num_cores=2, num_subcores=NSUB)
    agg = jax.ShapeDtypeStruct((NCH * NPAD, CW), jnp.bfloat16)
    deg = jax.ShapeDtypeStruct((DEG_ROWS,), jnp.float32)
    f = pl.kernel(
        _sc_body,
        out_type=[agg, agg, agg, deg, deg, deg],
        mesh=mesh,
        compiler_params=pltpu.CompilerParams(use_tc_tiling_on_sc=False),
        scratch_types=[
            pltpu.VMEM_SHARED((ACC_ROWS, CW), jnp.bfloat16),
            pltpu.VMEM_SHARED((DEG_ROWS,), jnp.float32),
            pltpu.VMEM_SHARED((NPAD, CW), jnp.bfloat16),
            pltpu.VMEM((4 * MICRO, CW), jnp.bfloat16),
            pltpu.VMEM((QROWS, MICRO), jnp.int32),
            pltpu.VMEM((QROWS, MICRO), jnp.int32),
            pltpu.VMEM((144,), jnp.float32),
            pltpu.VMEM((144,), jnp.float32),
            pltpu.VMEM((MICRO,), jnp.float32),
            [pltpu.SemaphoreType.DMA] * 4,
            [pltpu.SemaphoreType.DMA] * 4,
            [pltpu.SemaphoreType.DMA] * 4,
            pltpu.SemaphoreType.DMA,
        ],
    )
    return f(yui, yiu, yuu, sui, siu, suu, dui, diu, duu)


def kernel(x_user, x_item, ei_user_item, ei_item_user, ei_user_user,
           W_src_ui, W_dst_ui, b_ui,
           W_src_iu, W_dst_iu, b_iu,
           W_src_uu, W_dst_uu, b_uu):
    yui, yiu, yuu = _y_matmuls(x_user, x_item, W_src_ui, W_src_iu, W_src_uu)

    def prep_src(ei):
        return jnp.concatenate(
            [ei[0], jnp.zeros((EPAD - E,), jnp.int32)]).reshape(IDROWS, MICRO)

    def prep_dst(ei):
        return jnp.concatenate(
            [ei[1], jnp.full((EPAD - E,), N, jnp.int32)]).reshape(IDROWS, MICRO)

    sui, dui = prep_src(ei_user_item), prep_dst(ei_user_item)
    siu, diu = prep_src(ei_item_user), prep_dst(ei_item_user)
    suu, duu = prep_src(ei_user_user), prep_dst(ei_user_user)

    aui, aiu, auu, dgui, dgiu, dguu = _sc_gather_scatter(
        yui.reshape(NCH * NPAD, CW), yiu.reshape(NCH * NPAD, CW),
        yuu.reshape(NCH * NPAD, CW), sui, siu, suu, dui, diu, duu)

    out_user, out_item = _combine(
        aui.reshape(NCH, NPAD, CW), aiu.reshape(NCH, NPAD, CW),
        auu.reshape(NCH, NPAD, CW),
        dgui.reshape(DEG_ROWS, 1), dgiu.reshape(DEG_ROWS, 1),
        dguu.reshape(DEG_ROWS, 1),
        x_user, x_item, W_dst_ui, W_dst_iu, W_dst_uu,
        b_ui.reshape(1, D), b_iu.reshape(1, D), b_uu.reshape(1, D))
    return out_user, out_item


# speedup vs baseline: 1.6013x; 1.0195x over previous
import functools

import jax
import jax.numpy as jnp
from jax import lax
from jax.experimental import pallas as pl
from jax.experimental.pallas import tpu as pltpu
from jax.experimental.pallas import tpu_sc as plsc

N = 50000
D = 128
E = 500000
CW = 32
NCH = D // CW
EPAD = 524288
NSUB = 16
EPT = EPAD // NSUB
MICRO = 128
QROWS = 64
IDROWS = EPAD // MICRO
TROWS = EPT // MICRO
NPAD = 50048
ACC_ROWS = NPAD
WPT = NPAD // NSUB
DEG_ROWS = NPAD
DPT = DEG_ROWS // NSUB
BM = 2000


def _y_body(xu_ref, xi_ref, wui_ref, wiu_ref, wuu_ref, yui_ref, yiu_ref, yuu_ref):
    xu = xu_ref[...]
    xi = xi_ref[...]
    yui = jnp.dot(xu, wui_ref[...], preferred_element_type=jnp.float32)
    yiu = jnp.dot(xi, wiu_ref[...], preferred_element_type=jnp.float32)
    yuu = jnp.dot(xu, wuu_ref[...], preferred_element_type=jnp.float32)
    for c in range(NCH):
        yui_ref[c] = yui[:, c * CW:(c + 1) * CW].astype(jnp.bfloat16)
        yiu_ref[c] = yiu[:, c * CW:(c + 1) * CW].astype(jnp.bfloat16)
        yuu_ref[c] = yuu[:, c * CW:(c + 1) * CW].astype(jnp.bfloat16)


def _y_matmuls(x_user, x_item, w_ui, w_iu, w_uu):
    xspec = pl.BlockSpec((BM, D), lambda r: (r, 0))
    wspec = pl.BlockSpec((D, D), lambda r: (0, 0))
    yspec = pl.BlockSpec((NCH, BM, CW), lambda r: (0, r, 0))
    yshape = jax.ShapeDtypeStruct((NCH, NPAD, CW), jnp.bfloat16)
    return pl.pallas_call(
        _y_body,
        grid=(N // BM,),
        in_specs=[xspec, xspec, wspec, wspec, wspec],
        out_specs=[yspec, yspec, yspec],
        out_shape=[yshape, yshape, yshape],
    )(x_user, x_item, w_ui, w_iu, w_uu)


def _combine_body(aui_ref, aiu_ref, auu_ref, dui_ref, diu_ref, duu_ref,
                  xu_ref, xi_ref, wdui_ref, wdiu_ref, wduu_ref,
                  bui_ref, biu_ref, buu_ref, ou_ref, oi_ref):
    rui = 1.0 / jnp.maximum(dui_ref[...], 1.0)
    riu = 1.0 / jnp.maximum(diu_ref[...], 1.0)
    ruu = 1.0 / jnp.maximum(duu_ref[...], 1.0)
    xu = xu_ref[...]
    xi = xi_ref[...]
    zu = (jnp.dot(xu, wdiu_ref[...], preferred_element_type=jnp.float32)
          + jnp.dot(xu, wduu_ref[...], preferred_element_type=jnp.float32)
          + biu_ref[...] + buu_ref[...])
    zi = (jnp.dot(xi, wdui_ref[...], preferred_element_type=jnp.float32)
          + bui_ref[...])
    au = jnp.concatenate(
        [aiu_ref[c].astype(jnp.float32) * riu
         + auu_ref[c].astype(jnp.float32) * ruu for c in range(NCH)], axis=1)
    ai = jnp.concatenate(
        [aui_ref[c].astype(jnp.float32) * rui for c in range(NCH)], axis=1)
    ou_ref[...] = au + zu
    oi_ref[...] = ai + zi


def _combine(aui, aiu, auu, dgui, dgiu, dguu, x_user, x_item,
             wdui, wdiu, wduu, bui, biu, buu):
    aspec = pl.BlockSpec((NCH, BM, CW), lambda r: (0, r, 0))
    dspec = pl.BlockSpec((BM, 1), lambda r: (r, 0))
    xspec = pl.BlockSpec((BM, D), lambda r: (r, 0))
    wspec = pl.BlockSpec((D, D), lambda r: (0, 0))
    bspec = pl.BlockSpec((1, D), lambda r: (0, 0))
    ospec = pl.BlockSpec((BM, D), lambda r: (r, 0))
    oshape = jax.ShapeDtypeStruct((N, D), jnp.float32)
    return pl.pallas_call(
        _combine_body,
        grid=(N // BM,),
        in_specs=[aspec, aspec, aspec, dspec, dspec, dspec,
                  xspec, xspec, wspec, wspec, wspec, bspec, bspec, bspec],
        out_specs=[ospec, ospec],
        out_shape=[oshape, oshape],
    )(aui, aiu, auu, dgui, dgiu, dguu, x_user, x_item,
      wdui, wdiu, wduu, bui, biu, buu)


def _sc_body(yui, yiu, yuu, sui, siu, suu, dui, diu, duu,
             aui, aiu, auu, dgui, dgiu, dguu,
             acc, sdeg, tab, rows, sidx, didx, degstage, zdeg, ones,
             gsems, ssems, dsems, wsem):
    cid = lax.axis_index("c")
    sid = lax.axis_index("s")
    z16 = jnp.zeros((16,), jnp.float32)
    o16 = jnp.ones((16,), jnp.float32)

    def zd(i, c):
        zdeg[pl.ds(i * 16, 16)] = z16
        return c
    lax.fori_loop(0, zdeg.shape[0] // 16, zd, 0)
    for j in range(MICRO // 16):
        ones[pl.ds(j * 16, 16)] = o16

    rowsA = rows.at[pl.ds(0, MICRO)]
    rowsB = rows.at[pl.ds(MICRO, MICRO)]

    def run_type(yref, sref, dref, aggref, degref, deg_core):
        def pass_body(p, carry):
            ch = 2 * cid + p
            do_deg = jnp.logical_and(cid == deg_core, p == 0)
            plsc.subcore_barrier()

            sbase = ch * NPAD + sid * WPT
            tbase = sid * WPT

            def stg(k, c):
                pltpu.sync_copy(yref.at[pl.ds(sbase + 512 * k, 512)], rows)
                pltpu.sync_copy(rows, tab.at[pl.ds(tbase + 512 * k, 512)])
                return c
            lax.fori_loop(0, WPT // 512, stg, 0)
            pltpu.sync_copy(yref.at[pl.ds(sbase + WPT - WPT % 512, WPT % 512)],
                            rows.at[pl.ds(0, WPT % 512)])
            pltpu.sync_copy(rows.at[pl.ds(0, WPT % 512)],
                            tab.at[pl.ds(tbase + WPT - WPT % 512, WPT % 512)])

            zb16 = jnp.zeros((32,), jnp.bfloat16)

            def zr(i, c):
                rows[i, pl.ds(0, 32)] = zb16
                return c
            lax.fori_loop(0, MICRO, zr, 0)
            zb = sid * WPT
            zcps = [pltpu.async_copy(
                        rowsA, acc.at[pl.ds(zb + MICRO * k, MICRO)], wsem)
                    for k in range(WPT // MICRO)]
            zcps.append(pltpu.async_copy(
                rows.at[pl.ds(0, WPT % MICRO)],
                acc.at[pl.ds(zb + WPT - WPT % MICRO, WPT % MICRO)], wsem))

            @pl.when(do_deg)
            def _():
                def zdg(k, c):
                    pltpu.sync_copy(zdeg.at[pl.ds(0, 136)],
                                    sdeg.at[pl.ds(sid * DPT + 136 * k, 136)])
                    return c
                lax.fori_loop(0, DPT // 136, zdg, 0)
            for cp in zcps:
                cp.wait()
            plsc.subcore_barrier()

            dbase = sid * TROWS

            def quarter(q, cq):
                pltpu.sync_copy(sref.at[pl.ds(dbase + q * QROWS, QROWS)], sidx)
                pltpu.sync_copy(dref.at[pl.ds(dbase + q * QROWS, QROWS)], didx)

                def slot(j):
                    return rows.at[pl.ds(MICRO * j, MICRO)]

                def fire_g(t, j):
                    pltpu.async_copy(tab.at[sidx.at[t]], slot(j), gsems[j])

                def wait_g(t, j):
                    pltpu.make_async_copy(tab.at[sidx.at[t]], slot(j),
                                          gsems[j]).wait()

                def fire_s(t, j):
                    pltpu.async_copy(slot(j), acc.at[didx.at[t]], ssems[j],
                                     add=True)

                    @pl.when(do_deg)
                    def _():
                        pltpu.async_copy(ones, sdeg.at[didx.at[t]], dsems[j],
                                         add=True)

                def wait_s(t, j):
                    pltpu.make_async_copy(slot(j), acc.at[didx.at[t]],
                                          ssems[j]).wait()

                    @pl.when(do_deg)
                    def _():
                        pltpu.make_async_copy(ones, sdeg.at[didx.at[t]],
                                              dsems[j]).wait()

                fire_g(0, 0)
                fire_g(1, 1)
                fire_g(2, 2)

                def step(u, cu):
                    t0 = 4 * u
                    for j in range(4):
                        t = t0 + j

                        @pl.when(t >= 1)
                        def _(t=t, j=j):
                            wait_s(t - 1, (j + 3) % 4)

                        @pl.when(t + 3 < QROWS)
                        def _(t=t, j=j):
                            fire_g(t + 3, (j + 3) % 4)
                        wait_g(t, j)
                        fire_s(t, j)
                    return cu
                lax.fori_loop(0, QROWS // 4, step, 0)
                wait_s(QROWS - 1, 3)
                return cq
            lax.fori_loop(0, TROWS // QROWS, quarter, 0)
            plsc.subcore_barrier()

            wb = sid * WPT

            def wp(k, c):
                @pl.when(k > 0)
                def _():
                    pltpu.make_async_copy(
                        rowsA, aggref.at[pl.ds(0, MICRO)], wsem).wait()
                    pltpu.make_async_copy(
                        rowsB, aggref.at[pl.ds(0, MICRO)], wsem).wait()
                pltpu.sync_copy(acc.at[pl.ds(wb + MICRO * 2 * k, MICRO)],
                                rowsA)
                pltpu.async_copy(
                    rowsA,
                    aggref.at[pl.ds(ch * NPAD + wb + MICRO * 2 * k, MICRO)],
                    wsem)
                pltpu.sync_copy(acc.at[pl.ds(wb + MICRO * (2 * k + 1), MICRO)],
                                rowsB)
                pltpu.async_copy(
                    rowsB,
                    aggref.at[pl.ds(ch * NPAD + wb + MICRO * (2 * k + 1),
                                    MICRO)],
                    wsem)
                return c
            lax.fori_loop(0, WPT // (2 * MICRO), wp, 0)
            pltpu.make_async_copy(rowsA, aggref.at[pl.ds(0, MICRO)],
                                  wsem).wait()
            pltpu.make_async_copy(rowsB, aggref.at[pl.ds(0, MICRO)],
                                  wsem).wait()
            pltpu.sync_copy(acc.at[pl.ds(wb + WPT - WPT % MICRO, WPT % MICRO)],
                            rows.at[pl.ds(0, WPT % MICRO)])
            pltpu.sync_copy(rows.at[pl.ds(0, WPT % MICRO)],
                            aggref.at[pl.ds(ch * NPAD + wb + WPT - WPT % MICRO,
                                            WPT % MICRO)])

            @pl.when(do_deg)
            def _():
                def wdg(k, c):
                    pltpu.sync_copy(sdeg.at[pl.ds(sid * DPT + 136 * k, 136)],
                                    degstage.at[pl.ds(0, 136)])
                    pltpu.sync_copy(degstage.at[pl.ds(0, 136)],
                                    degref.at[pl.ds(sid * DPT + 136 * k, 136)])
                    return c
                lax.fori_loop(0, DPT // 136, wdg, 0)
            return carry
        lax.fori_loop(0, 2, pass_body, 0)

    run_type(yui, sui, dui, aui, dgui, 0)
    run_type(yiu, siu, diu, aiu, dgiu, 1)
    run_type(yuu, suu, duu, auu, dguu, 1)


def _sc_gather_scatter(yui, yiu, yuu, sui, siu, suu, dui, diu, duu):
    mesh = plsc.VectorSubcoreMesh(core_axis_name="c", subcore_axis_name="s",
                                  num_cores=2, num_subcores=NSUB)
    agg = jax.ShapeDtypeStruct((NCH * NPAD, CW), jnp.bfloat16)
    deg = jax.ShapeDtypeStruct((DEG_ROWS,), jnp.float32)
    f = pl.kernel(
        _sc_body,
        out_type=[agg, agg, agg, deg, deg, deg],
        mesh=mesh,
        compiler_params=pltpu.CompilerParams(use_tc_tiling_on_sc=False),
        scratch_types=[
            pltpu.VMEM_SHARED((ACC_ROWS, CW), jnp.bfloat16),
            pltpu.VMEM_SHARED((DEG_ROWS,), jnp.float32),
            pltpu.VMEM_SHARED((NPAD, CW), jnp.bfloat16),
            pltpu.VMEM((4 * MICRO, CW), jnp.bfloat16),
            pltpu.VMEM((QROWS, MICRO), jnp.int32),
            pltpu.VMEM((QROWS, MICRO), jnp.int32),
            pltpu.VMEM((144,), jnp.float32),
            pltpu.VMEM((144,), jnp.float32),
            pltpu.VMEM((MICRO,), jnp.float32),
            [pltpu.SemaphoreType.DMA] * 4,
            [pltpu.SemaphoreType.DMA] * 4,
            [pltpu.SemaphoreType.DMA] * 4,
            pltpu.SemaphoreType.DMA,
        ],
    )
    return f(yui, yiu, yuu, sui, siu, suu, dui, diu, duu)


def kernel(x_user, x_item, ei_user_item, ei_item_user, ei_user_user,
           W_src_ui, W_dst_ui, b_ui,
           W_src_iu, W_dst_iu, b_iu,
           W_src_uu, W_dst_uu, b_uu):
    yui, yiu, yuu = _y_matmuls(x_user, x_item, W_src_ui, W_src_iu, W_src_uu)

    def prep_src(ei):
        return jnp.concatenate(
            [ei[0], jnp.zeros((EPAD - E,), jnp.int32)]).reshape(IDROWS, MICRO)

    def prep_dst(ei):
        return jnp.concatenate(
            [ei[1], jnp.full((EPAD - E,), N, jnp.int32)]).reshape(IDROWS, MICRO)

    sui, dui = prep_src(ei_user_item), prep_dst(ei_user_item)
    siu, diu = prep_src(ei_item_user), prep_dst(ei_item_user)
    suu, duu = prep_src(ei_user_user), prep_dst(ei_user_user)

    aui, aiu, auu, dgui, dgiu, dguu = _sc_gather_scatter(
        yui.reshape(NCH * NPAD, CW), yiu.reshape(NCH * NPAD, CW),
        yuu.reshape(NCH * NPAD, CW), sui, siu, suu, dui, diu, duu)

    out_user, out_item = _combine(
        aui.reshape(NCH, NPAD, CW), aiu.reshape(NCH, NPAD, CW),
        auu.reshape(NCH, NPAD, CW),
        dgui.reshape(DEG_ROWS, 1), dgiu.reshape(DEG_ROWS, 1),
        dguu.reshape(DEG_ROWS, 1),
        x_user, x_item, W_dst_ui, W_dst_iu, W_dst_uu,
        b_ui.reshape(1, D), b_iu.reshape(1, D), b_uu.reshape(1, D))
    return out_user, out_item
